# Initial kernel scaffold; baseline (speedup 1.0000x reference)
import jax
import jax.numpy as jnp
from jax.experimental import pallas as pl


def kernel(x, embedding):
    raise NotImplementedError("write your pallas kernel here")


# speedup vs baseline: 1.2966x; 1.2966x over previous
import functools

import jax
import jax.numpy as jnp
from jax import lax
from jax.experimental import pallas as pl
from jax.experimental.pallas import tpu as pltpu
from jax.experimental.pallas import tpu_sc as plsc

_BATCH, _HIST, _DIM = 16384, 50, 32
_B = _BATCH * _HIST

_info = plsc.get_sparse_core_info()
_NC, _NS = _info.num_cores, _info.num_subcores
_NW = _NC * _NS
_BPW = _B // _NW
_G = 128
_NG = _BPW // _G
_K = 8
_NCH = _NG // _K

_mesh = plsc.VectorSubcoreMesh(core_axis_name="c", subcore_axis_name="s")


@functools.partial(
    pl.kernel,
    mesh=_mesh,
    out_type=jax.ShapeDtypeStruct((_NW * _NG, _G, _DIM), jnp.float32),
    scratch_types=[
        pltpu.VMEM((_NG, _G), jnp.int32),
        pltpu.VMEM((_K, _G, _DIM), jnp.float32),
        pltpu.SemaphoreType.DMA,
    ],
    compiler_params=pltpu.CompilerParams(use_tc_tiling_on_sc=False),
)
def _emb_gather(idx_hbm, table_hbm, out_hbm, idx_v, rows_v, sem):
    wid = lax.axis_index("s") * _NC + lax.axis_index("c")
    gbase = wid * _NG
    pltpu.sync_copy(idx_hbm.at[pl.ds(gbase, _NG)], idx_v)

    def chunk(c, carry):
        g0 = c * _K
        copies = [
            pltpu.async_copy(table_hbm.at[idx_v.at[g0 + j]], rows_v.at[j], sem)
            for j in range(_K)
        ]
        for cp in copies:
            cp.wait()
        pltpu.sync_copy(rows_v, out_hbm.at[pl.ds(gbase + g0, _K)])
        return carry

    lax.fori_loop(0, _NCH, chunk, 0)


def kernel(x, embedding):
    idx = x.reshape(_NW * _NG, _G)
    out = _emb_gather(idx, embedding)
    return out.reshape(_BATCH, _HIST, _DIM)


# speedup vs baseline: 1.3109x; 1.0110x over previous
import functools

import jax
import jax.numpy as jnp
from jax import lax
from jax.experimental import pallas as pl
from jax.experimental.pallas import tpu as pltpu
from jax.experimental.pallas import tpu_sc as plsc

_BATCH, _HIST, _DIM = 16384, 50, 32
_B = _BATCH * _HIST

_info = plsc.get_sparse_core_info()
_NC, _NS = _info.num_cores, _info.num_subcores
_NW = _NC * _NS
_BPW = _B // _NW
_G = 128
_NG = _BPW // _G
_K = 8
_NCH = _NG // _K

_mesh = plsc.VectorSubcoreMesh(core_axis_name="c", subcore_axis_name="s")


@functools.partial(
    pl.kernel,
    mesh=_mesh,
    out_type=jax.ShapeDtypeStruct((_NW * _NG, _G, _DIM), jnp.float32),
    scratch_types=[
        pltpu.VMEM((_NG, _G), jnp.int32),
        pltpu.VMEM((2, _K, _G, _DIM), jnp.float32),
        pltpu.SemaphoreType.DMA,
        pltpu.SemaphoreType.DMA,
    ],
    compiler_params=pltpu.CompilerParams(use_tc_tiling_on_sc=False),
)
def _emb_gather(idx_hbm, table_hbm, out_hbm, idx_v, rows_v, gsem, osem):
    wid = lax.axis_index("s") * _NC + lax.axis_index("c")
    gbase = wid * _NG
    pltpu.sync_copy(idx_hbm.at[pl.ds(gbase, _NG)], idx_v)

    def step(c, carry):
        b = c % 2

        @pl.when(c >= 2)
        def _wait_writeout():
            pltpu.make_async_copy(
                rows_v.at[b], out_hbm.at[pl.ds(gbase, _K)], osem
            ).wait()

        @pl.when(c < _NCH)
        def _fire_gathers():
            g0 = c * _K
            for j in range(_K):
                pltpu.async_copy(
                    table_hbm.at[idx_v.at[g0 + j]], rows_v.at[b].at[j], gsem
                )

        @pl.when(c >= 1)
        def _drain_and_writeout():
            pltpu.make_async_copy(
                out_hbm.at[pl.ds(gbase, _K)], rows_v.at[1 - b], gsem
            ).wait()
            pltpu.async_copy(
                rows_v.at[1 - b], out_hbm.at[pl.ds(gbase + (c - 1) * _K, _K)], osem
            )

        return carry

    lax.fori_loop(0, _NCH + 1, step, 0)
    pltpu.make_async_copy(
        rows_v.at[(_NCH - 1) % 2], out_hbm.at[pl.ds(gbase, _K)], osem
    ).wait()


def kernel(x, embedding):
    idx = x.reshape(_NW * _NG, _G)
    out = _emb_gather(idx, embedding)
    return out.reshape(_BATCH, _HIST, _DIM)


# speedup vs baseline: 1.8062x; 1.3779x over previous
import functools

import jax
import jax.numpy as jnp
from jax import lax
from jax.experimental import pallas as pl
from jax.experimental.pallas import tpu as pltpu
from jax.experimental.pallas import tpu_sc as plsc

_BATCH, _HIST, _DIM = 16384, 50, 32

_info = plsc.get_sparse_core_info()
_NC, _NS = _info.num_cores, _info.num_subcores
_NW = _NC * _NS
_RPW = _BATCH // _NW
_K = 16
_NCH = _RPW // _K

_mesh = plsc.VectorSubcoreMesh(core_axis_name="c", subcore_axis_name="s")


@functools.partial(
    pl.kernel,
    mesh=_mesh,
    out_type=jax.ShapeDtypeStruct((_BATCH, _HIST, _DIM), jnp.float32),
    scratch_types=[
        pltpu.VMEM((_RPW, _HIST), jnp.int32),
        pltpu.VMEM((2, _K, _HIST, _DIM), jnp.float32),
        pltpu.SemaphoreType.DMA,
        pltpu.SemaphoreType.DMA,
    ],
    compiler_params=pltpu.CompilerParams(use_tc_tiling_on_sc=False),
)
def _emb_gather(idx_hbm, table_hbm, out_hbm, idx_v, rows_v, gsem, osem):
    wid = lax.axis_index("s") * _NC + lax.axis_index("c")
    base = wid * _RPW
    pltpu.sync_copy(idx_hbm.at[pl.ds(base, _RPW)], idx_v)

    def step(c, carry):
        b = c % 2

        @pl.when(c >= 2)
        def _wait_writeout():
            pltpu.make_async_copy(
                rows_v.at[b], out_hbm.at[pl.ds(base, _K)], osem
            ).wait()

        @pl.when(c < _NCH)
        def _fire_gathers():
            r0 = c * _K
            for j in range(_K):
                pltpu.async_copy(
                    table_hbm.at[idx_v.at[r0 + j]], rows_v.at[b].at[j], gsem
                )

        @pl.when(c >= 1)
        def _drain_and_writeout():
            pltpu.make_async_copy(
                out_hbm.at[pl.ds(base, _K)], rows_v.at[1 - b], gsem
            ).wait()
            pltpu.async_copy(
                rows_v.at[1 - b],
                out_hbm.at[pl.ds(base + (c - 1) * _K, _K)],
                osem,
            )

        return carry

    lax.fori_loop(0, _NCH + 1, step, 0)
    pltpu.make_async_copy(
        rows_v.at[(_NCH - 1) % 2], out_hbm.at[pl.ds(base, _K)], osem
    ).wait()


def kernel(x, embedding):
    return _emb_gather(x, embedding)


# speedup vs baseline: 1.8655x; 1.0328x over previous
import functools

import jax
import jax.numpy as jnp
from jax import lax
from jax.experimental import pallas as pl
from jax.experimental.pallas import tpu as pltpu
from jax.experimental.pallas import tpu_sc as plsc

_BATCH, _HIST, _DIM = 16384, 50, 32

_info = plsc.get_sparse_core_info()
_NC, _NS = _info.num_cores, _info.num_subcores
_NW = _NC * _NS
_BPW = _BATCH // _NW
_NJ = _BPW // 128
_HSLAB = (_DIM // 8) * (_BATCH // 128) * 8 * 128
_RSLAB = (_BATCH // 128) * 8 * 128

_mesh = plsc.VectorSubcoreMesh(core_axis_name="c", subcore_axis_name="s")


@functools.partial(
    pl.kernel,
    mesh=_mesh,
    out_type=jax.ShapeDtypeStruct((_HIST * _HSLAB,), jnp.float32),
    scratch_types=[
        pltpu.VMEM((_HIST, _BPW), jnp.int32),
        pltpu.VMEM((2, _BPW, _DIM), jnp.float32),
        pltpu.VMEM((2, _NJ * 128 * _DIM), jnp.float32),
        pltpu.SemaphoreType.DMA,
        pltpu.SemaphoreType.DMA,
    ],
    compiler_params=pltpu.CompilerParams(use_tc_tiling_on_sc=False, needs_layout_passes=False),
)
def _emb_gather(xt_hbm, table_hbm, out_hbm, idx_v, gbuf, tbuf, gsem, osem):
    wid = lax.axis_index("s") * _NC + lax.axis_index("c")
    b0 = wid * _BPW
    pltpu.sync_copy(xt_hbm.at[:, pl.ds(b0, _BPW)], idx_v)

    iota16 = lax.iota(jnp.int32, 16)
    basec_lo = (iota16 // 8) * (_NJ * 8 * 128) + (iota16 % 8) * 128
    basec_hi = basec_lo + 2 * (_NJ * 8 * 128)

    def step(h, carry):
        b2 = (h - 1) % 2

        @pl.when(h < _HIST)
        def _fire_gathers():
            for j in range(_NJ):
                pltpu.async_copy(
                    table_hbm.at[idx_v.at[h, pl.ds(j * 128, 128)]],
                    gbuf.at[h % 2, pl.ds(j * 128, 128)],
                    gsem,
                )

        @pl.when(h >= 1)
        def _transpose_and_writeout():
            hh = h - 1
            for j in range(_NJ):
                pltpu.make_async_copy(
                    table_hbm.at[pl.ds(0, 128)],
                    gbuf.at[b2, pl.ds(j * 128, 128)],
                    gsem,
                ).wait()

            @pl.when(h >= 3)
            def _wait_writeout():
                pltpu.make_async_copy(
                    out_hbm.at[pl.ds(0, _NJ * 128 * _DIM)], tbuf.at[b2], osem
                ).wait()

            def tbody(i, c2):
                for t in range(8):
                    jb = i * 8 + t
                    scal = jb + (jb // 128) * (1024 - 128)
                    lo = gbuf.at[b2, jb, pl.ds(0, 16)][...]
                    hi = gbuf.at[b2, jb, pl.ds(16, 16)][...]
                    plsc.store_scatter(tbuf.at[b2], [basec_lo + scal], lo)
                    plsc.store_scatter(tbuf.at[b2], [basec_hi + scal], hi)
                return c2

            lax.fori_loop(0, _BPW // 8, tbody, 0)

            obase = hh * _HSLAB + wid * (_NJ * 1024)
            for r in range(_DIM // 8):
                pltpu.async_copy(
                    tbuf.at[b2, pl.ds(r * (_NJ * 1024), _NJ * 1024)],
                    out_hbm.at[pl.ds(obase + r * _RSLAB, _NJ * 1024)],
                    osem,
                )

        return carry

    lax.fori_loop(0, _HIST + 1, step, 0)
    for b2 in (0, 1):
        pltpu.make_async_copy(
            out_hbm.at[pl.ds(0, _NJ * 128 * _DIM)], tbuf.at[b2], osem
        ).wait()


def kernel(x, embedding):
    out = _emb_gather(x.T, embedding)
    out = out.reshape(_HIST, _DIM // 8, _BATCH // 128, 8, 128)
    return out.transpose(2, 4, 0, 1, 3).reshape(_BATCH, _HIST, _DIM)
